# baseline (speedup 1.0000x reference)
import functools

import jax
import jax.numpy as jnp
from jax import lax
from jax.experimental import pallas as pl
from jax.experimental.pallas import tpu as pltpu
from jax.experimental.pallas import tpu_sc as plsc

N1 = 5000
N2 = 5000
D = 128
H = 128
E = 160000
P = 100000
HALF = 5120
NT = 2 * HALF
NTILE = 16
NSC = 2
ECH = 128
ENCH = 80
PCH = 128
PNCH = 25
STRIPE = HALF // NTILE

_MESH = plsc.VectorSubcoreMesh(
    core_axis_name="c", subcore_axis_name="s", num_cores=NSC,
    num_subcores=NTILE)


def _mm(a, b):
  return lax.dot_general(a, b, (((1,), (0,)), ((), ())),
                         precision=lax.Precision.HIGHEST,
                         preferred_element_type=jnp.float32)


def _k1_body(x1, x2, w1, b1, w2, b2, w0, cnt, g_out, dinv_out):
  dinv = lax.rsqrt(cnt[...][:, 0:1])
  dinv_out[...] = dinv
  h1 = jnp.maximum(_mm(x1[...], w1[...]) + b1[...], 0.0)
  h2 = jnp.maximum(_mm(x2[...], w2[...]) + b2[...], 0.0)
  zeros8 = jnp.zeros((HALF - N1, H), jnp.float32)
  g_out[0:N1, :] = dinv[0:N1] * _mm(h1, w0[...])
  g_out[N1:HALF, :] = zeros8
  g_out[HALF:HALF + N2, :] = dinv[HALF:HALF + N2] * _mm(h2, w0[...])
  g_out[HALF + N2:NT, :] = zeros8


def _kmid_body(agg, dinv, w, b, g_out, x_out):
  x = dinv[...] * agg[...] + b[...]
  x_out[...] = x
  g_out[...] = dinv[...] * _mm(x, w[...])


def _klast_body(agg, dinv, b, x_out):
  x_out[...] = dinv[...] * agg[...] + b[...]


def _reduce_body(psum, out):
  row = lax.broadcasted_iota(jnp.int32, (H, 8), 0)
  col = lax.broadcasted_iota(jnp.int32, (H, 8), 1)
  sel = (row // 16 == col).astype(jnp.float32)
  out[...] = _mm(psum[...], sel)


def _tc_reduce(psum):
  return pl.pallas_call(
      _reduce_body,
      out_shape=jax.ShapeDtypeStruct((psum.shape[0], 8), jnp.float32),
  )(psum)


def _tc_k1(x1, x2, w1, b1, w2, b2, w0, cnt):
  return pl.pallas_call(
      _k1_body,
      out_shape=(jax.ShapeDtypeStruct((NT, H), jnp.float32),
                 jax.ShapeDtypeStruct((NT, 1), jnp.float32)),
  )(x1, x2, w1, b1, w2, b2, w0, cnt)


def _tc_mid(agg, dinv, w, b):
  return pl.pallas_call(
      _kmid_body,
      out_shape=(jax.ShapeDtypeStruct((NT, H), jnp.float32),
                 jax.ShapeDtypeStruct((NT, H), jnp.float32)),
  )(agg, dinv, w, b)


def _tc_last(agg, dinv, b):
  return pl.pallas_call(
      _klast_body,
      out_shape=jax.ShapeDtypeStruct((NT, H), jnp.float32),
  )(agg, dinv, b)


NBUF = 4


def _agg_body(g_hbm, gidx_hbm, sidx_hbm, out_hbm,
              idxg_v, idxs_v, rows, gsems, ssems, accum_sh):
  c = lax.axis_index("c")
  s = lax.axis_index("s")
  pltpu.sync_copy(gidx_hbm.at[c, s], idxg_v)
  pltpu.sync_copy(sidx_hbm.at[c, s], idxs_v)
  pltpu.sync_copy(g_hbm.at[pl.ds(c * HALF + s * STRIPE, STRIPE)],
                  accum_sh.at[pl.ds(s * STRIPE, STRIPE)])
  plsc.subcore_barrier()

  for b in range(NBUF):
    pltpu.async_copy(g_hbm.at[idxg_v.at[b]], rows[b], gsems[b])

  def rnd(q, carry):
    j0 = q * NBUF
    for b in range(NBUF):
      pltpu.make_async_copy(g_hbm.at[idxg_v.at[j0 + b]],
                            rows[b], gsems[b]).wait()
      pltpu.async_copy(rows[b], accum_sh.at[idxs_v.at[j0 + b]], ssems[b],
                       add=True)
    for b in range(NBUF):
      pltpu.make_async_copy(rows[b], accum_sh.at[idxs_v.at[j0 + b]],
                            ssems[b]).wait()
      nxt = j0 + NBUF + b
      nxt = jnp.where(nxt < ENCH, nxt, b)
      pltpu.async_copy(g_hbm.at[idxg_v.at[nxt]], rows[b], gsems[b])
    return carry

  lax.fori_loop(0, ENCH // NBUF, rnd, 0)
  for b in range(NBUF):
    pltpu.make_async_copy(g_hbm.at[idxg_v.at[b]], rows[b], gsems[b]).wait()
  plsc.subcore_barrier()
  pltpu.sync_copy(accum_sh.at[pl.ds(s * STRIPE, STRIPE)],
                  out_hbm.at[pl.ds(c * HALF + s * STRIPE, STRIPE)])


@functools.partial(
    pl.kernel, mesh=_MESH,
    out_type=jax.ShapeDtypeStruct((NT, H), jnp.float32),
    scratch_types=[
        pltpu.VMEM((ENCH, ECH), jnp.int32),
        pltpu.VMEM((ENCH, ECH), jnp.int32),
        [pltpu.VMEM((ECH, H), jnp.float32) for _ in range(NBUF)],
        [pltpu.SemaphoreType.DMA for _ in range(NBUF)],
        [pltpu.SemaphoreType.DMA for _ in range(NBUF)],
        pltpu.VMEM_SHARED((HALF, H), jnp.float32),
    ])
def _sc_agg(g_hbm, gidx_hbm, sidx_hbm, out_hbm,
            idxg_v, idxs_v, rows, gsems, ssems, accum_sh):
  _agg_body(g_hbm, gidx_hbm, sidx_hbm, out_hbm,
            idxg_v, idxs_v, rows, gsems, ssems, accum_sh)


def _score_body(x3_hbm, x5_hbm, mi_hbm, di_hbm, out_hbm,
                mi_v, di_v, a3, a5, b3, b5, obuf, sem):
  c = lax.axis_index("c")
  s = lax.axis_index("s")
  pltpu.sync_copy(mi_hbm.at[c, s], mi_v)
  pltpu.sync_copy(di_hbm.at[c, s], di_v)

  def chunk(j, carry):
    d0 = pltpu.async_copy(x3_hbm.at[mi_v.at[j]], a3, sem)
    d1 = pltpu.async_copy(x5_hbm.at[mi_v.at[j]], a5, sem)
    d2 = pltpu.async_copy(x3_hbm.at[di_v.at[j]], b3, sem)
    d3 = pltpu.async_copy(x5_hbm.at[di_v.at[j]], b5, sem)
    d0.wait()
    d1.wait()
    d2.wait()
    d3.wait()

    def group(gi, carry2):
      for i2 in range(16):
        row = gi * 16 + i2
        acc = jnp.zeros((16,), jnp.float32)
        for k in range(H // 16):
          sl = pl.ds(k * 16, 16)
          acc = acc + a3[row, sl] * b3[row, sl]
          acc = acc + a5[row, sl] * b5[row, sl]
        obuf[row] = acc
      return carry2

    lax.fori_loop(0, PCH // 16, group, 0)
    w = c * NTILE + s
    pltpu.sync_copy(obuf, out_hbm.at[pl.ds(w * (PNCH * PCH) + j * PCH, PCH)])
    return carry

  lax.fori_loop(0, PNCH, chunk, 0)


@functools.partial(
    pl.kernel, mesh=_MESH,
    out_type=jax.ShapeDtypeStruct((NSC * NTILE * PNCH * PCH, 16), jnp.float32),
    scratch_types=[
        pltpu.VMEM((PNCH, PCH), jnp.int32),
        pltpu.VMEM((PNCH, PCH), jnp.int32),
        pltpu.VMEM((PCH, H), jnp.float32),
        pltpu.VMEM((PCH, H), jnp.float32),
        pltpu.VMEM((PCH, H), jnp.float32),
        pltpu.VMEM((PCH, H), jnp.float32),
        pltpu.VMEM((PCH, 16), jnp.float32),
        pltpu.SemaphoreType.DMA,
    ])
def _sc_score(x3_hbm, x5_hbm, mi_hbm, di_hbm, out_hbm,
              mi_v, di_v, a3, a5, b3, b5, obuf, sem):
  _score_body(x3_hbm, x5_hbm, mi_hbm, di_hbm, out_hbm,
              mi_v, di_v, a3, a5, b3, b5, obuf, sem)


def kernel(x_n1, x_n2, lin_n1_W, lin_n1_b, lin_n2_W, lin_n2_b, gcn_W, gcn_b,
           edge_index_het, edge_index):
  src = edge_index_het[0].astype(jnp.int32)
  dst = edge_index_het[1].astype(jnp.int32)

  epad = NTILE * ENCH * ECH
  padg = jnp.zeros((epad - E,), jnp.int32)
  pads = jnp.full((epad - E,), N1, jnp.int32)
  gidx = jnp.stack([
      jnp.concatenate([dst + HALF, padg]).reshape(NTILE, ENCH, ECH),
      jnp.concatenate([src, padg]).reshape(NTILE, ENCH, ECH)])
  sidx = jnp.stack([
      jnp.concatenate([src, pads]).reshape(NTILE, ENCH, ECH),
      jnp.concatenate([dst, pads]).reshape(NTILE, ENCH, ECH)])

  m = edge_index[0].astype(jnp.int32)
  di = edge_index[1].astype(jnp.int32)
  ppad = NSC * NTILE * PNCH * PCH
  padp = jnp.zeros((ppad - P,), jnp.int32)
  mi = jnp.concatenate([m, padp]).reshape(NSC, NTILE, PNCH, PCH)
  didx = jnp.concatenate([di + HALF, padp]).reshape(NSC, NTILE, PNCH, PCH)

  b1 = lin_n1_b.reshape(1, H)
  b2 = lin_n2_b.reshape(1, H)

  counts = _sc_agg(jnp.ones((NT, H), jnp.float32), gidx, sidx)
  g, dinv = _tc_k1(x_n1, x_n2, lin_n1_W, b1, lin_n2_W, b2, gcn_W[0], counts)
  a = _sc_agg(g, gidx, sidx)
  g, _ = _tc_mid(a, dinv, gcn_W[1], gcn_b[0].reshape(1, H))
  a = _sc_agg(g, gidx, sidx)
  g, x3 = _tc_mid(a, dinv, gcn_W[2], gcn_b[1].reshape(1, H))
  a = _sc_agg(g, gidx, sidx)
  g, _ = _tc_mid(a, dinv, gcn_W[3], gcn_b[2].reshape(1, H))
  a = _sc_agg(g, gidx, sidx)
  x5 = _tc_last(a, dinv, gcn_b[3].reshape(1, H))

  psum = _sc_score(x3, x5, mi, didx)
  scored = _tc_reduce(psum.reshape(ppad * 16 // 128, 128))
  return scored.reshape(ppad, 1)[:P]


# speedup vs baseline: 1.1194x; 1.1194x over previous
import functools

import jax
import jax.numpy as jnp
from jax import lax
from jax.experimental import pallas as pl
from jax.experimental.pallas import tpu as pltpu
from jax.experimental.pallas import tpu_sc as plsc

N1 = 5000
N2 = 5000
D = 128
H = 128
E = 160000
P = 100000
HALF = 5120
NT = 2 * HALF
NTILE = 16
NSC = 2
ECH = 128
ENCH = 80
PCH = 128
PNCH = 25
STRIPE = HALF // NTILE

_MESH = plsc.VectorSubcoreMesh(
    core_axis_name="c", subcore_axis_name="s", num_cores=NSC,
    num_subcores=NTILE)


def _mm(a, b):
  return lax.dot_general(a, b, (((1,), (0,)), ((), ())),
                         precision=lax.Precision.HIGHEST,
                         preferred_element_type=jnp.float32)


def _k1_body(x1, x2, w1, b1, w2, b2, w0, cnt, g_out, dinv_out):
  dinv = lax.rsqrt(cnt[...][:, 0:1])
  dinv_out[...] = dinv
  h1 = jnp.maximum(_mm(x1[...], w1[...]) + b1[...], 0.0)
  h2 = jnp.maximum(_mm(x2[...], w2[...]) + b2[...], 0.0)
  zeros8 = jnp.zeros((HALF - N1, H), jnp.float32)
  g_out[0:N1, :] = dinv[0:N1] * _mm(h1, w0[...])
  g_out[N1:HALF, :] = zeros8
  g_out[HALF:HALF + N2, :] = dinv[HALF:HALF + N2] * _mm(h2, w0[...])
  g_out[HALF + N2:NT, :] = zeros8


def _kmid_body(agg, dinv, w, b, g_out, x_out):
  x = dinv[...] * agg[...] + b[...]
  x_out[...] = x
  g_out[...] = dinv[...] * _mm(x, w[...])


def _klast_body(agg, dinv, b, x_out):
  x_out[...] = dinv[...] * agg[...] + b[...]


def _reduce_body(psum, out):
  row = lax.broadcasted_iota(jnp.int32, (H, 8), 0)
  col = lax.broadcasted_iota(jnp.int32, (H, 8), 1)
  sel = (row // 16 == col).astype(jnp.float32)
  out[...] = _mm(psum[...], sel)


def _tc_reduce(psum):
  return pl.pallas_call(
      _reduce_body,
      out_shape=jax.ShapeDtypeStruct((psum.shape[0], 8), jnp.float32),
  )(psum)


def _tc_k1(x1, x2, w1, b1, w2, b2, w0, cnt):
  return pl.pallas_call(
      _k1_body,
      out_shape=(jax.ShapeDtypeStruct((NT, H), jnp.float32),
                 jax.ShapeDtypeStruct((NT, 1), jnp.float32)),
  )(x1, x2, w1, b1, w2, b2, w0, cnt)


def _tc_mid(agg, dinv, w, b):
  return pl.pallas_call(
      _kmid_body,
      out_shape=(jax.ShapeDtypeStruct((NT, H), jnp.float32),
                 jax.ShapeDtypeStruct((NT, H), jnp.float32)),
  )(agg, dinv, w, b)


def _tc_last(agg, dinv, b):
  return pl.pallas_call(
      _klast_body,
      out_shape=jax.ShapeDtypeStruct((NT, H), jnp.float32),
  )(agg, dinv, b)


NBUF = 4


def _agg_body(g_hbm, gidx_hbm, sidx_hbm, out_hbm,
              idxg_v, idxs_v, rows, gsems, accum_sh):
  c = lax.axis_index("c")
  s = lax.axis_index("s")
  pltpu.sync_copy(gidx_hbm.at[c, s], idxg_v)
  pltpu.sync_copy(sidx_hbm.at[c, s], idxs_v)
  pltpu.sync_copy(g_hbm.at[pl.ds(c * HALF + s * STRIPE, STRIPE)],
                  accum_sh.at[pl.ds(s * STRIPE, STRIPE)])
  plsc.subcore_barrier()

  for b in range(NBUF):
    pltpu.async_copy(g_hbm.at[idxg_v.at[b]], rows[b], gsems[b])

  def rnd(q, carry):
    j0 = q * NBUF
    for b in range(NBUF):
      pltpu.make_async_copy(g_hbm.at[idxg_v.at[j0 + b]],
                            rows[b], gsems[b]).wait()
      pltpu.sync_copy(rows[b], accum_sh.at[idxs_v.at[j0 + b]], add=True)
      nxt = j0 + NBUF + b
      nxt = jnp.where(nxt < ENCH, nxt, b)
      pltpu.async_copy(g_hbm.at[idxg_v.at[nxt]], rows[b], gsems[b])
    return carry

  lax.fori_loop(0, ENCH // NBUF, rnd, 0)
  for b in range(NBUF):
    pltpu.make_async_copy(g_hbm.at[idxg_v.at[b]], rows[b], gsems[b]).wait()
  plsc.subcore_barrier()
  pltpu.sync_copy(accum_sh.at[pl.ds(s * STRIPE, STRIPE)],
                  out_hbm.at[pl.ds(c * HALF + s * STRIPE, STRIPE)])


@functools.partial(
    pl.kernel, mesh=_MESH,
    out_type=jax.ShapeDtypeStruct((NT, H), jnp.float32),
    scratch_types=[
        pltpu.VMEM((ENCH, ECH), jnp.int32),
        pltpu.VMEM((ENCH, ECH), jnp.int32),
        [pltpu.VMEM((ECH, H), jnp.float32) for _ in range(NBUF)],
        [pltpu.SemaphoreType.DMA for _ in range(NBUF)],
        pltpu.VMEM_SHARED((HALF, H), jnp.float32),
    ])
def _sc_agg(g_hbm, gidx_hbm, sidx_hbm, out_hbm,
            idxg_v, idxs_v, rows, gsems, accum_sh):
  _agg_body(g_hbm, gidx_hbm, sidx_hbm, out_hbm,
            idxg_v, idxs_v, rows, gsems, accum_sh)


def _score_body(x3_hbm, x5_hbm, mi_hbm, di_hbm, out_hbm,
                mi_v, di_v, bufs, sems, obuf):
  c = lax.axis_index("c")
  s = lax.axis_index("s")
  pltpu.sync_copy(mi_hbm.at[c, s], mi_v)
  pltpu.sync_copy(di_hbm.at[c, s], di_v)
  nsub = PNCH * 2

  def _idx(kk, which):
    v = mi_v if which == 0 else di_v
    return v.at[kk // 2, pl.ds((kk % 2) * 64, 64)]

  def _start(st, kk):
    a3, a5, b3, b5 = bufs[st]
    s0, s1, s2, s3 = sems[st]
    pltpu.async_copy(x3_hbm.at[_idx(kk, 0)], a3, s0)
    pltpu.async_copy(x5_hbm.at[_idx(kk, 0)], a5, s1)
    pltpu.async_copy(x3_hbm.at[_idx(kk, 1)], b3, s2)
    pltpu.async_copy(x5_hbm.at[_idx(kk, 1)], b5, s3)

  def _wait(st, kk):
    a3, a5, b3, b5 = bufs[st]
    s0, s1, s2, s3 = sems[st]
    pltpu.make_async_copy(x3_hbm.at[_idx(kk, 0)], a3, s0).wait()
    pltpu.make_async_copy(x5_hbm.at[_idx(kk, 0)], a5, s1).wait()
    pltpu.make_async_copy(x3_hbm.at[_idx(kk, 1)], b3, s2).wait()
    pltpu.make_async_copy(x5_hbm.at[_idx(kk, 1)], b5, s3).wait()

  def _compute(st, kk):
    a3, a5, b3, b5 = bufs[st]

    def group(gi, carry2):
      for i2 in range(16):
        row = gi * 16 + i2
        acc = jnp.zeros((16,), jnp.float32)
        for k in range(H // 16):
          sl = pl.ds(k * 16, 16)
          acc = acc + a3[row, sl] * b3[row, sl]
          acc = acc + a5[row, sl] * b5[row, sl]
        obuf[kk * 8 + gi * 2 + i2 // 8, pl.ds((i2 % 8) * 16, 16)] = acc
      return carry2

    lax.fori_loop(0, 4, group, 0)

  _start(0, 0)
  _start(1, 1)

  def rnd(q, carry):
    for st in range(2):
      kk = 2 * q + st
      _wait(st, kk)
      _compute(st, kk)
      nk = 2 * q + st + 2
      nk = jnp.where(nk < nsub, nk, st)
      _start(st, nk)
    return carry

  lax.fori_loop(0, nsub // 2, rnd, 0)
  for st in range(2):
    _wait(st, st)
  w = c * NTILE + s
  pltpu.sync_copy(obuf, out_hbm.at[pl.ds(w * (PNCH * PCH * 16 // 128),
                                         PNCH * PCH * 16 // 128)])


@functools.partial(
    pl.kernel, mesh=_MESH,
    out_type=jax.ShapeDtypeStruct((NSC * NTILE * PNCH * PCH // 8, H),
                                   jnp.float32),
    scratch_types=[
        pltpu.VMEM((PNCH, PCH), jnp.int32),
        pltpu.VMEM((PNCH, PCH), jnp.int32),
        [[pltpu.VMEM((64, H), jnp.float32) for _ in range(4)]
         for _ in range(2)],
        [[pltpu.SemaphoreType.DMA for _ in range(4)] for _ in range(2)],
        pltpu.VMEM((PNCH * PCH * 16 // 128, H), jnp.float32),
    ])
def _sc_score(x3_hbm, x5_hbm, mi_hbm, di_hbm, out_hbm,
              mi_v, di_v, bufs, sems, obuf):
  _score_body(x3_hbm, x5_hbm, mi_hbm, di_hbm, out_hbm,
              mi_v, di_v, bufs, sems, obuf)


def kernel(x_n1, x_n2, lin_n1_W, lin_n1_b, lin_n2_W, lin_n2_b, gcn_W, gcn_b,
           edge_index_het, edge_index):
  src = edge_index_het[0].astype(jnp.int32)
  dst = edge_index_het[1].astype(jnp.int32)

  epad = NTILE * ENCH * ECH
  padg = jnp.zeros((epad - E,), jnp.int32)
  pads = jnp.full((epad - E,), N1, jnp.int32)
  gidx = jnp.stack([
      jnp.concatenate([dst + HALF, padg]).reshape(NTILE, ENCH, ECH),
      jnp.concatenate([src, padg]).reshape(NTILE, ENCH, ECH)])
  sidx = jnp.stack([
      jnp.concatenate([src, pads]).reshape(NTILE, ENCH, ECH),
      jnp.concatenate([dst, pads]).reshape(NTILE, ENCH, ECH)])

  m = edge_index[0].astype(jnp.int32)
  di = edge_index[1].astype(jnp.int32)
  ppad = NSC * NTILE * PNCH * PCH
  padp = jnp.zeros((ppad - P,), jnp.int32)
  mi = jnp.concatenate([m, padp]).reshape(NSC, NTILE, PNCH, PCH)
  didx = jnp.concatenate([di + HALF, padp]).reshape(NSC, NTILE, PNCH, PCH)

  b1 = lin_n1_b.reshape(1, H)
  b2 = lin_n2_b.reshape(1, H)

  counts = _sc_agg(jnp.ones((NT, H), jnp.float32), gidx, sidx)
  g, dinv = _tc_k1(x_n1, x_n2, lin_n1_W, b1, lin_n2_W, b2, gcn_W[0], counts)
  a = _sc_agg(g, gidx, sidx)
  g, _ = _tc_mid(a, dinv, gcn_W[1], gcn_b[0].reshape(1, H))
  a = _sc_agg(g, gidx, sidx)
  g, x3 = _tc_mid(a, dinv, gcn_W[2], gcn_b[1].reshape(1, H))
  a = _sc_agg(g, gidx, sidx)
  g, _ = _tc_mid(a, dinv, gcn_W[3], gcn_b[2].reshape(1, H))
  a = _sc_agg(g, gidx, sidx)
  x5 = _tc_last(a, dinv, gcn_b[3].reshape(1, H))

  psum = _sc_score(x3, x5, mi, didx)
  scored = _tc_reduce(psum)
  return scored.reshape(ppad, 1)[:P]
